# Initial kernel scaffold; baseline (speedup 1.0000x reference)
import jax
import jax.numpy as jnp
from jax.experimental import pallas as pl


def kernel(exercises, categories, exercise_embed, category_embed, position_embed):
    raise NotImplementedError("write your pallas kernel here")


# speedup vs baseline: 3.6036x; 3.6036x over previous
import functools

import jax
import jax.numpy as jnp
from jax import lax
from jax.experimental import pallas as pl
from jax.experimental.pallas import tpu as pltpu
from jax.experimental.pallas import tpu_sc as plsc

_N_DIMS = 64
_SEQ_LEN = 200
_BATCH = 4096
_ROWS = _BATCH * _SEQ_LEN
_NW = 32
_RPW = _ROWS // _NW
_CH = 128
_NCH = _RPW // _CH
_P2 = 2 * _SEQ_LEN

_mesh = plsc.VectorSubcoreMesh(core_axis_name="c", subcore_axis_name="s")


@functools.partial(
    pl.kernel,
    mesh=_mesh,
    out_type=jax.ShapeDtypeStruct((_ROWS, _N_DIMS), jnp.float32),
    scratch_types=[
        pltpu.VMEM((_CH,), jnp.int32),
        pltpu.VMEM((_CH,), jnp.int32),
        pltpu.VMEM((_CH, _N_DIMS), jnp.float32),
        pltpu.VMEM((_CH, _N_DIMS), jnp.float32),
        pltpu.VMEM((_P2, _N_DIMS), jnp.float32),
        pltpu.SemaphoreType.DMA,
        pltpu.SemaphoreType.DMA,
    ],
    compiler_params=pltpu.CompilerParams(use_tc_tiling_on_sc=False),
)
def _embed_kernel(eidx_hbm, cidx_hbm, etab_hbm, ctab_hbm, ptab_hbm, out_hbm,
                  eidx_v, cidx_v, erows_v, crows_v, pos_v, sem_e, sem_c):
    wid = lax.axis_index("s") * 2 + lax.axis_index("c")
    base = wid * _RPW
    pltpu.sync_copy(ptab_hbm, pos_v)

    def chunk_body(ci, _):
        off = base + ci * _CH
        pltpu.sync_copy(eidx_hbm.at[pl.ds(off, _CH)], eidx_v)
        pltpu.sync_copy(cidx_hbm.at[pl.ds(off, _CH)], cidx_v)
        cp_e = pltpu.async_copy(etab_hbm.at[eidx_v], erows_v, sem_e)
        cp_c = pltpu.async_copy(ctab_hbm.at[cidx_v], crows_v, sem_c)
        cp_e.wait()
        cp_c.wait()
        pbase = off % _SEQ_LEN

        def row_body(r, _):
            pr = pbase + r
            for g in range(_N_DIMS // 16):
                sl = pl.ds(g * 16, 16)
                erows_v[r, sl] = erows_v[r, sl] + crows_v[r, sl] + pos_v[pr, sl]
            return 0

        lax.fori_loop(0, _CH, row_body, 0)
        pltpu.sync_copy(erows_v, out_hbm.at[pl.ds(off, _CH)])
        return 0

    lax.fori_loop(0, _NCH, chunk_body, 0)


def kernel(exercises, categories, exercise_embed, category_embed, position_embed):
    eidx = exercises.reshape(-1).astype(jnp.int32)
    cidx = categories.reshape(-1).astype(jnp.int32)
    pos2 = jnp.concatenate([position_embed, position_embed], axis=0)
    out = _embed_kernel(eidx, cidx, exercise_embed, category_embed, pos2)
    return out.reshape(_BATCH, _SEQ_LEN, _N_DIMS)


# speedup vs baseline: 4.9025x; 1.3604x over previous
import functools

import jax
import jax.numpy as jnp
from jax import lax
from jax.experimental import pallas as pl
from jax.experimental.pallas import tpu as pltpu
from jax.experimental.pallas import tpu_sc as plsc

_N_DIMS = 64
_SEQ_LEN = 200
_BATCH = 4096
_ROWS = _BATCH * _SEQ_LEN
_NW = 32
_RPW = _ROWS // _NW
_CH = 128
_NCH = _RPW // _CH
_P2 = 2 * _SEQ_LEN

_mesh = plsc.VectorSubcoreMesh(core_axis_name="c", subcore_axis_name="s")


@functools.partial(
    pl.kernel,
    mesh=_mesh,
    out_type=jax.ShapeDtypeStruct((_ROWS, _N_DIMS), jnp.float32),
    scratch_types=[
        pltpu.VMEM((_NCH, _CH), jnp.int32),
        pltpu.VMEM((_NCH, _CH), jnp.int32),
        pltpu.VMEM((2, _CH, _N_DIMS), jnp.float32),
        pltpu.VMEM((2, _CH, _N_DIMS), jnp.float32),
        pltpu.VMEM((_P2, _N_DIMS), jnp.float32),
        pltpu.SemaphoreType.DMA,
        pltpu.SemaphoreType.DMA,
        pltpu.SemaphoreType.DMA,
        pltpu.SemaphoreType.DMA,
        pltpu.SemaphoreType.DMA,
        pltpu.SemaphoreType.DMA,
    ],
    compiler_params=pltpu.CompilerParams(use_tc_tiling_on_sc=False),
)
def _embed_kernel(eidx_hbm, cidx_hbm, etab_hbm, ctab_hbm, ptab_hbm, out_hbm,
                  eidx_v, cidx_v, erows_v, crows_v, pos_v,
                  sem_ge0, sem_ge1, sem_gc0, sem_gc1, sem_o0, sem_o1):
    sem_ge = (sem_ge0, sem_ge1)
    sem_gc = (sem_gc0, sem_gc1)
    sem_o = (sem_o0, sem_o1)
    wid = lax.axis_index("s") * 2 + lax.axis_index("c")
    base = wid * _RPW
    pltpu.sync_copy(ptab_hbm, pos_v)
    pltpu.sync_copy(eidx_hbm.at[pl.ds(wid * _NCH, _NCH)], eidx_v)
    pltpu.sync_copy(cidx_hbm.at[pl.ds(wid * _NCH, _NCH)], cidx_v)

    def gather_start(c, b):
        pltpu.async_copy(etab_hbm.at[eidx_v.at[c]], erows_v.at[b], sem_ge[b])
        pltpu.async_copy(ctab_hbm.at[cidx_v.at[c]], crows_v.at[b], sem_gc[b])

    def gather_wait(b):
        pltpu.make_async_copy(
            etab_hbm.at[eidx_v.at[0]], erows_v.at[b], sem_ge[b]).wait()
        pltpu.make_async_copy(
            ctab_hbm.at[cidx_v.at[0]], crows_v.at[b], sem_gc[b]).wait()

    def out_wait(b):
        pltpu.make_async_copy(
            erows_v.at[b], out_hbm.at[pl.ds(0, _CH)], sem_o[b]).wait()

    gather_start(0, 0)

    def outer(half, _):
        cb = half * 2
        for b in range(2):
            c = cb + b
            nb = 1 - b

            @pl.when(c < _NCH - 1)
            def _():
                @pl.when(c >= 1)
                def _():
                    out_wait(nb)

                gather_start(c + 1, nb)

            gather_wait(b)
            pbase = (c * _CH) % _SEQ_LEN

            def row_body(r, _):
                pr = pbase + r
                for g in range(_N_DIMS // 16):
                    sl = pl.ds(g * 16, 16)
                    erows_v[b, r, sl] = (erows_v[b, r, sl] + crows_v[b, r, sl]
                                         + pos_v[pr, sl])
                return 0

            lax.fori_loop(0, _CH, row_body, 0)
            pltpu.async_copy(
                erows_v.at[b], out_hbm.at[pl.ds(base + c * _CH, _CH)], sem_o[b])
        return 0

    lax.fori_loop(0, _NCH // 2, outer, 0)
    out_wait(0)
    out_wait(1)


def kernel(exercises, categories, exercise_embed, category_embed, position_embed):
    eidx = exercises.reshape(_ROWS // _CH, _CH).astype(jnp.int32)
    cidx = categories.reshape(_ROWS // _CH, _CH).astype(jnp.int32)
    pos2 = jnp.concatenate([position_embed, position_embed], axis=0)
    out = _embed_kernel(eidx, cidx, exercise_embed, category_embed, pos2)
    return out.reshape(_BATCH, _SEQ_LEN, _N_DIMS)


# speedup vs baseline: 5.2226x; 1.0653x over previous
import functools

import jax
import jax.numpy as jnp
from jax import lax
from jax.experimental import pallas as pl
from jax.experimental.pallas import tpu as pltpu
from jax.experimental.pallas import tpu_sc as plsc

_N_DIMS = 64
_SEQ_LEN = 200
_BATCH = 4096
_ROWS = _BATCH * _SEQ_LEN
_NW = 32
_RPW = _ROWS // _NW
_CH = 128
_NCH = _RPW // _CH
_OCH = _CH // 2
_OROWS = _ROWS // 2
_PROWS = 168

_mesh = plsc.VectorSubcoreMesh(core_axis_name="c", subcore_axis_name="s")


@functools.partial(
    pl.kernel,
    mesh=_mesh,
    out_type=jax.ShapeDtypeStruct((_OROWS, 128), jnp.float32),
    scratch_types=[
        pltpu.VMEM((_NCH, _CH), jnp.int32),
        pltpu.VMEM((_NCH, _CH), jnp.int32),
        pltpu.VMEM((2, _CH, _N_DIMS), jnp.float32),
        pltpu.VMEM((2, _CH, _N_DIMS), jnp.float32),
        pltpu.VMEM((_PROWS, 128), jnp.float32),
        pltpu.VMEM((2, _OCH, 128), jnp.float32),
        pltpu.SemaphoreType.DMA,
        pltpu.SemaphoreType.DMA,
        pltpu.SemaphoreType.DMA,
        pltpu.SemaphoreType.DMA,
        pltpu.SemaphoreType.DMA,
        pltpu.SemaphoreType.DMA,
    ],
    compiler_params=pltpu.CompilerParams(use_tc_tiling_on_sc=False),
)
def _embed_kernel(eidx_hbm, cidx_hbm, etab_hbm, ctab_hbm, ptab_hbm, out_hbm,
                  eidx_v, cidx_v, erows_v, crows_v, pos_v, obuf_v,
                  sem_ge0, sem_ge1, sem_gc0, sem_gc1, sem_o0, sem_o1):
    sem_ge = (sem_ge0, sem_ge1)
    sem_gc = (sem_gc0, sem_gc1)
    sem_o = (sem_o0, sem_o1)
    wid = lax.axis_index("s") * 2 + lax.axis_index("c")
    base = wid * _RPW
    obase = wid * (_RPW // 2)
    pltpu.sync_copy(ptab_hbm, pos_v.at[pl.ds(0, _SEQ_LEN // 2)])
    pltpu.sync_copy(ptab_hbm.at[pl.ds(0, _PROWS - _SEQ_LEN // 2)],
                    pos_v.at[pl.ds(_SEQ_LEN // 2, _PROWS - _SEQ_LEN // 2)])
    pltpu.sync_copy(eidx_hbm.at[pl.ds(wid * _NCH, _NCH)], eidx_v)
    pltpu.sync_copy(cidx_hbm.at[pl.ds(wid * _NCH, _NCH)], cidx_v)

    def gather_start(c, b):
        pltpu.async_copy(etab_hbm.at[eidx_v.at[c]], erows_v.at[b], sem_ge[b])
        pltpu.async_copy(ctab_hbm.at[cidx_v.at[c]], crows_v.at[b], sem_gc[b])

    def gather_wait(b):
        pltpu.make_async_copy(
            etab_hbm.at[eidx_v.at[0]], erows_v.at[b], sem_ge[b]).wait()
        pltpu.make_async_copy(
            ctab_hbm.at[cidx_v.at[0]], crows_v.at[b], sem_gc[b]).wait()

    def out_wait(b):
        pltpu.make_async_copy(
            obuf_v.at[b], out_hbm.at[pl.ds(0, _OCH)], sem_o[b]).wait()

    gather_start(0, 0)

    def outer(half, _):
        cb = half * 2
        for b in range(2):
            c = cb + b
            nb = 1 - b

            @pl.when(c < _NCH - 1)
            def _():
                gather_start(c + 1, nb)

            gather_wait(b)

            @pl.when(c >= 2)
            def _():
                out_wait(b)

            pb2 = ((c * _CH) % _SEQ_LEN) // 2

            def row_body(q, _):
                prow = pb2 + q
                for h in range(8):
                    r = 2 * q + h // 4
                    g = (h % 4) * 16
                    oslice = pl.ds(h * 16, 16)
                    obuf_v[b, q, oslice] = (erows_v[b, r, pl.ds(g, 16)]
                                            + crows_v[b, r, pl.ds(g, 16)]
                                            + pos_v[prow, oslice])
                return 0

            lax.fori_loop(0, _OCH, row_body, 0)
            pltpu.async_copy(
                obuf_v.at[b], out_hbm.at[pl.ds(obase + c * _OCH, _OCH)],
                sem_o[b])
        return 0

    lax.fori_loop(0, _NCH // 2, outer, 0)
    out_wait(0)
    out_wait(1)


def kernel(exercises, categories, exercise_embed, category_embed, position_embed):
    eidx = exercises.reshape(_ROWS // _CH, _CH).astype(jnp.int32)
    cidx = categories.reshape(_ROWS // _CH, _CH).astype(jnp.int32)
    pos128 = position_embed.reshape(_SEQ_LEN // 2, 128)
    out = _embed_kernel(eidx, cidx, exercise_embed, category_embed, pos128)
    return out.reshape(_BATCH, _SEQ_LEN, _N_DIMS)


# speedup vs baseline: 6.5637x; 1.2568x over previous
import functools

import jax
import jax.numpy as jnp
from jax import lax
from jax.experimental import pallas as pl
from jax.experimental.pallas import tpu as pltpu
from jax.experimental.pallas import tpu_sc as plsc

_N_DIMS = 64
_SEQ_LEN = 200
_BATCH = 4096
_ROWS = _BATCH * _SEQ_LEN
_NW = 32
_RPW = _ROWS // _NW
_CH = 128
_NCH = _RPW // _CH
_OCH = _CH // 2
_OROWS = _ROWS // 2
_PROWS = 168

_mesh = plsc.VectorSubcoreMesh(core_axis_name="c", subcore_axis_name="s")


@functools.partial(
    pl.kernel,
    mesh=_mesh,
    out_type=jax.ShapeDtypeStruct((_OROWS, 128), jnp.float32),
    scratch_types=[
        pltpu.VMEM((_NCH, _CH), jnp.int32),
        pltpu.VMEM((_NCH, _CH), jnp.int32),
        pltpu.VMEM((2, _CH, _N_DIMS), jnp.float32),
        pltpu.VMEM((2, _CH, _N_DIMS), jnp.float32),
        pltpu.VMEM((_PROWS, 128), jnp.float32),
        pltpu.VMEM((2, _OCH, 128), jnp.float32),
        pltpu.SemaphoreType.DMA,
        pltpu.SemaphoreType.DMA,
        pltpu.SemaphoreType.DMA,
        pltpu.SemaphoreType.DMA,
        pltpu.SemaphoreType.DMA,
        pltpu.SemaphoreType.DMA,
    ],
    compiler_params=pltpu.CompilerParams(use_tc_tiling_on_sc=False),
)
def _embed_kernel(eidx_hbm, cidx_hbm, etab_hbm, ctab_hbm, ptab_hbm, out_hbm,
                  eidx_v, cidx_v, erows_v, crows_v, pos_v, obuf_v,
                  sem_ge0, sem_ge1, sem_gc0, sem_gc1, sem_o0, sem_o1):
    sem_ge = (sem_ge0, sem_ge1)
    sem_gc = (sem_gc0, sem_gc1)
    sem_o = (sem_o0, sem_o1)
    wid = lax.axis_index("s") * 2 + lax.axis_index("c")
    base = wid * _RPW
    obase = wid * (_RPW // 2)
    pltpu.sync_copy(ptab_hbm, pos_v.at[pl.ds(0, _SEQ_LEN // 2)])
    pltpu.sync_copy(ptab_hbm.at[pl.ds(0, _PROWS - _SEQ_LEN // 2)],
                    pos_v.at[pl.ds(_SEQ_LEN // 2, _PROWS - _SEQ_LEN // 2)])
    pltpu.sync_copy(eidx_hbm.at[pl.ds(wid * _NCH, _NCH)], eidx_v)
    pltpu.sync_copy(cidx_hbm.at[pl.ds(wid * _NCH, _NCH)], cidx_v)

    def gather_start(c, b):
        pltpu.async_copy(etab_hbm.at[eidx_v.at[c]], erows_v.at[b], sem_ge[b])
        pltpu.async_copy(ctab_hbm.at[cidx_v.at[c]], crows_v.at[b], sem_gc[b])

    def gather_wait(b):
        pltpu.make_async_copy(
            etab_hbm.at[eidx_v.at[0]], erows_v.at[b], sem_ge[b]).wait()
        pltpu.make_async_copy(
            ctab_hbm.at[cidx_v.at[0]], crows_v.at[b], sem_gc[b]).wait()

    def out_wait(b):
        pltpu.make_async_copy(
            obuf_v.at[b], out_hbm.at[pl.ds(0, _OCH)], sem_o[b]).wait()

    gather_start(0, 0)

    def outer(half, _):
        cb = half * 2
        for b in range(2):
            c = cb + b
            nb = 1 - b

            @pl.when(c < _NCH - 1)
            def _():
                gather_start(c + 1, nb)

            gather_wait(b)

            @pl.when(c >= 2)
            def _():
                out_wait(b)

            pb2 = ((c * _CH) % _SEQ_LEN) // 2

            @plsc.parallel_loop(0, _OCH, unroll=4)
            def row_body(q):
                prow = pb2 + q
                for h in range(8):
                    r = 2 * q + h // 4
                    g = (h % 4) * 16
                    oslice = pl.ds(h * 16, 16)
                    obuf_v[b, q, oslice] = (erows_v[b, r, pl.ds(g, 16)]
                                            + crows_v[b, r, pl.ds(g, 16)]
                                            + pos_v[prow, oslice])
            pltpu.async_copy(
                obuf_v.at[b], out_hbm.at[pl.ds(obase + c * _OCH, _OCH)],
                sem_o[b])
        return 0

    lax.fori_loop(0, _NCH // 2, outer, 0)
    out_wait(0)
    out_wait(1)


def kernel(exercises, categories, exercise_embed, category_embed, position_embed):
    eidx = exercises.reshape(_ROWS // _CH, _CH).astype(jnp.int32)
    cidx = categories.reshape(_ROWS // _CH, _CH).astype(jnp.int32)
    pos128 = position_embed.reshape(_SEQ_LEN // 2, 128)
    out = _embed_kernel(eidx, cidx, exercise_embed, category_embed, pos128)
    return out.reshape(_BATCH, _SEQ_LEN, _N_DIMS)
